# baseline (speedup 1.0000x reference)
import functools

import jax
import jax.numpy as jnp
from jax.experimental import pallas as pl
from jax.experimental.pallas import tpu as pltpu


def _layer_kernel(a_ref, x_ref, w_ref, b_ref, o_ref, s_ref):
    @pl.when(pl.program_id(0) == 0)
    def _():
        sup = jnp.dot(x_ref[...], w_ref[...], preferred_element_type=jnp.float32)
        s_ref[...] = sup.astype(s_ref.dtype)

    acc = jnp.dot(a_ref[...], s_ref[...], preferred_element_type=jnp.float32)
    o_ref[...] = jnp.maximum(acc + b_ref[...], 0.0).astype(o_ref.dtype)


def _gc_layer(a, x, w, b, out_dtype, block_m):
    n = a.shape[0]
    k = x.shape[1]
    f = w.shape[1]
    return pl.pallas_call(
        _layer_kernel,
        grid=(n // block_m,),
        in_specs=[
            pl.BlockSpec((block_m, n), lambda i: (i, 0)),
            pl.BlockSpec((n, k), lambda i: (0, 0)),
            pl.BlockSpec((k, f), lambda i: (0, 0)),
            pl.BlockSpec((1, f), lambda i: (0, 0)),
        ],
        out_specs=pl.BlockSpec((block_m, f), lambda i: (i, 0)),
        out_shape=jax.ShapeDtypeStruct((n, f), out_dtype),
        scratch_shapes=[pltpu.VMEM((n, f), jnp.bfloat16)],
    )(a, x, w, b)


@functools.partial(jax.jit, static_argnames=())
def kernel(x, adj, A2, W3, b3, W1, b1, W2, b2):
    bf = jnp.bfloat16
    xb = x.astype(bf)
    adjb = adj.astype(bf)
    a2b = A2.astype(bf)
    out1 = _gc_layer(adjb, xb, W3.astype(bf), b3.reshape(1, -1), bf, 256)
    out2 = _gc_layer(a2b, out1, W1.astype(bf), b1.reshape(1, -1), bf, 256)
    out3 = _gc_layer(a2b, out2, W2.astype(bf), b2.reshape(1, -1), jnp.float32, 256)
    return out3


# speedup vs baseline: 1.4874x; 1.4874x over previous
import jax
import jax.numpy as jnp
from jax.experimental import pallas as pl
from jax.experimental.pallas import tpu as pltpu

_P = jax.lax.Precision.DEFAULT


def _layer_kernel(a_ref, x_ref, w_ref, b_ref, o_ref, s_ref):
    @pl.when(pl.program_id(0) == 0)
    def _():
        s_ref[...] = jnp.dot(
            x_ref[...], w_ref[...],
            preferred_element_type=jnp.float32, precision=_P)

    acc = jnp.dot(
        a_ref[...], s_ref[...],
        preferred_element_type=jnp.float32, precision=_P)
    o_ref[...] = jnp.maximum(acc + b_ref[...], 0.0)


def _gc_layer(a, x, w, b, block_m):
    n = a.shape[0]
    k = x.shape[1]
    f = w.shape[1]
    return pl.pallas_call(
        _layer_kernel,
        grid=(n // block_m,),
        in_specs=[
            pl.BlockSpec((block_m, n), lambda i: (i, 0)),
            pl.BlockSpec((n, k), lambda i: (0, 0)),
            pl.BlockSpec((k, f), lambda i: (0, 0)),
            pl.BlockSpec((1, f), lambda i: (0, 0)),
        ],
        out_specs=pl.BlockSpec((block_m, f), lambda i: (i, 0)),
        out_shape=jax.ShapeDtypeStruct((n, f), jnp.float32),
        scratch_shapes=[pltpu.VMEM((n, f), jnp.float32)],
    )(a, x, w, b)


@jax.jit
def kernel(x, adj, A2, W3, b3, W1, b1, W2, b2):
    out1 = _gc_layer(adj, x, W3, b3.reshape(1, -1), 256)
    out2 = _gc_layer(A2, out1, W1, b1.reshape(1, -1), 256)
    out3 = _gc_layer(A2, out2, W2, b2.reshape(1, -1), 256)
    return out3
